# baseline (speedup 1.0000x reference)
import functools

import jax
import jax.numpy as jnp
from jax import lax
from jax.experimental import pallas as pl
from jax.experimental.pallas import tpu as pltpu
from jax.experimental.pallas import tpu_sc as plsc

EMB_DIM = 1024
LANES = 16
NUM_CORES = 2
NUM_SUBCORES = 16
NUM_WORKERS = NUM_CORES * NUM_SUBCORES


@jax.jit
def _lookup(seg_flat, table):
    n = seg_flat.shape[0]
    per_w = n // NUM_WORKERS
    pairs_w = per_w // 2
    n_groups = per_w // LANES
    mesh = plsc.VectorSubcoreMesh(core_axis_name="c", subcore_axis_name="s")

    @functools.partial(
        pl.kernel,
        out_type=jax.ShapeDtypeStruct((n // 2, 2 * EMB_DIM), jnp.float32),
        mesh=mesh,
        scratch_types=[
            pltpu.VMEM((per_w,), jnp.int32),
            pltpu.VMEM((9, 2 * EMB_DIM), jnp.float32),
            pltpu.SemaphoreType.DMA,
            pltpu.SemaphoreType.DMA,
        ],
    )
    def body(seg_hbm, table_hbm, out_hbm, idx_v, pair_v, stage_sem, sem):
        wid = lax.axis_index("s") * NUM_CORES + lax.axis_index("c")
        base = wid * per_w
        pbase = wid * pairs_w

        pltpu.async_copy(seg_hbm.at[pl.ds(base, per_w)], idx_v, stage_sem)
        for a in range(3):
            for b in range(3):
                pltpu.async_copy(
                    table_hbm.at[pl.ds(a, 1)], pair_v.at[pl.ds(3 * a + b, 1), pl.ds(0, EMB_DIM)], stage_sem
                )
                pltpu.async_copy(
                    table_hbm.at[pl.ds(b, 1)], pair_v.at[pl.ds(3 * a + b, 1), pl.ds(EMB_DIM, EMB_DIM)], stage_sem
                )
        pltpu.make_async_copy(
            seg_hbm.at[pl.ds(base, per_w)], idx_v, stage_sem
        ).wait()
        for a in range(3):
            for b in range(3):
                pltpu.make_async_copy(
                    table_hbm.at[pl.ds(a, 1)], pair_v.at[pl.ds(3 * a + b, 1), pl.ds(0, EMB_DIM)], stage_sem
                ).wait()
                pltpu.make_async_copy(
                    table_hbm.at[pl.ds(b, 1)], pair_v.at[pl.ds(3 * a + b, 1), pl.ds(EMB_DIM, EMB_DIM)], stage_sem
                ).wait()

        def group(g, carry):
            seg_vec = idx_v[pl.ds(g * LANES, LANES)]
            gbase = pbase + g * (LANES // 2)
            for i in range(LANES // 2):
                pid = seg_vec[2 * i] * 3 + seg_vec[2 * i + 1]
                pltpu.async_copy(
                    pair_v.at[pl.ds(pid, 1)], out_hbm.at[pl.ds(gbase + i, 1)], sem
                )
            return carry

        lax.fori_loop(0, n_groups, group, 0)

        pltpu.make_async_copy(
            out_hbm.at[pl.ds(pbase, pairs_w)],
            out_hbm.at[pl.ds(pbase, pairs_w)],
            sem,
        ).wait()

    return body(seg_flat, table)


def kernel(segment, table):
    b, s = segment.shape
    seg_flat = segment.reshape(b * s).astype(jnp.int32)
    out = _lookup(seg_flat, table)
    return out.reshape(b, s, EMB_DIM)


# speedup vs baseline: 3.4594x; 3.4594x over previous
import functools

import jax
import jax.numpy as jnp
from jax import lax
from jax.experimental import pallas as pl
from jax.experimental.pallas import tpu as pltpu
from jax.experimental.pallas import tpu_sc as plsc

EMB_DIM = 1024
LANES = 16
NUM_CORES = 2
NUM_SUBCORES = 16
NUM_WORKERS = NUM_CORES * NUM_SUBCORES


@jax.jit
def _lookup(seg_flat, table):
    n = seg_flat.shape[0]
    per_w = n // NUM_WORKERS
    n_groups = per_w // LANES
    mesh = plsc.VectorSubcoreMesh(core_axis_name="c", subcore_axis_name="s")

    @functools.partial(
        pl.kernel,
        out_type=jax.ShapeDtypeStruct((n, EMB_DIM), jnp.float32),
        mesh=mesh,
        scratch_types=[
            pltpu.VMEM((per_w,), jnp.int32),
            pltpu.VMEM((3, EMB_DIM), jnp.float32),
            pltpu.SemaphoreType.DMA,
        ],
    )
    def body(seg_hbm, table_hbm, out_hbm, idx_v, table_v, sem):
        wid = lax.axis_index("s") * NUM_CORES + lax.axis_index("c")
        base = wid * per_w
        pltpu.async_copy(table_hbm, table_v, sem)
        pltpu.async_copy(seg_hbm.at[pl.ds(base, per_w)], idx_v, sem)
        pltpu.make_async_copy(table_hbm, table_v, sem).wait()
        pltpu.make_async_copy(
            seg_hbm.at[pl.ds(base, per_w)], idx_v, sem
        ).wait()

        def group(g, carry):
            seg_vec = idx_v[pl.ds(g * LANES, LANES)]
            tok = base + g * LANES
            for r in range(LANES):
                pltpu.async_copy(
                    table_v.at[seg_vec[r]], out_hbm.at[tok + r], sem
                )
            return carry

        lax.fori_loop(0, n_groups, group, 0)

        pltpu.make_async_copy(
            out_hbm.at[pl.ds(base, per_w)],
            out_hbm.at[pl.ds(base, per_w)],
            sem,
        ).wait()

    return body(seg_flat, table)


def kernel(segment, table):
    b, s = segment.shape
    seg_flat = segment.reshape(b * s).astype(jnp.int32)
    out = _lookup(seg_flat, table)
    return out.reshape(b, s, EMB_DIM)
